# Initial kernel scaffold; baseline (speedup 1.0000x reference)
import jax
import jax.numpy as jnp
from jax.experimental import pallas as pl


def kernel(cell_feats, topo_feats, cell_mask, topo_mask, params):
    raise NotImplementedError("write your pallas kernel here")


# speedup vs baseline: 23.1881x; 23.1881x over previous
import jax
import jax.numpy as jnp
from jax.experimental import pallas as pl
from jax.experimental.pallas import tpu as pltpu

BS = 2
NC = 8192
NT = 512
FC = 16
FT = 10
H = 128
HEADS = 4
NPF = 32
NN = NC + NT
HH = HEADS * H
F32 = jnp.float32

CT_E = 512
CT_S = 512
NTILE = 512


def _pc(body, **kw):
    return pl.pallas_call(body, **kw)


def _lr(x):
    return jnp.where(x > 0, x, 0.2 * x)


def _hsel():
    d = jax.lax.broadcasted_iota(jnp.int32, (HH, HEADS), 0) // H
    hcol = jax.lax.broadcasted_iota(jnp.int32, (HH, HEADS), 1)
    return (d == hcol).astype(F32)


def _layernorm_relu_add(conv, x, g, b):
    mu = jnp.mean(conv, axis=1, keepdims=True)
    cmu = conv - mu
    var = jnp.mean(cmu * cmu, axis=1, keepdims=True)
    y = cmu * jax.lax.rsqrt(var + 1e-5) * g + b
    return x + jnp.maximum(y, 0.0)


def _embed_body(f_ref, w1_ref, b1_ref, w2_ref, b2_ref, emb_ref, pos_ref):
    f = f_ref[...]
    h = jnp.maximum(jnp.dot(f, w1_ref[...], preferred_element_type=F32) + b1_ref[...], 0.0)
    emb_ref[...] = jnp.dot(h, w2_ref[...], preferred_element_type=F32) + b2_ref[...]
    i = jax.lax.broadcasted_iota(jnp.int32, (1, NPF), 1).astype(F32)
    dim_t = jnp.exp((2.0 * jnp.floor(i / 2.0) / NPF) * jnp.log(10000.0))
    pe = f[:, 4:5] / dim_t
    pp = f[:, 5:6] / dim_t
    pos_ref[:, 0:NPF] = jnp.sin(pe)
    pos_ref[:, NPF:2 * NPF] = jnp.cos(pe)
    pos_ref[:, 2 * NPF:3 * NPF] = jnp.sin(pp)
    pos_ref[:, 3 * NPF:4 * NPF] = jnp.cos(pp)


def _embed(f2d, mp, tile):
    n, fdim = f2d.shape
    return _pc(
        _embed_body,
        grid=(n // tile,),
        in_specs=[
            pl.BlockSpec((tile, fdim), lambda i: (i, 0)),
            pl.BlockSpec((fdim, H), lambda i: (0, 0)),
            pl.BlockSpec((1, H), lambda i: (0, 0)),
            pl.BlockSpec((H, H), lambda i: (0, 0)),
            pl.BlockSpec((1, H), lambda i: (0, 0)),
        ],
        out_specs=[
            pl.BlockSpec((tile, H), lambda i: (i, 0)),
            pl.BlockSpec((tile, H), lambda i: (i, 0)),
        ],
        out_shape=[
            jax.ShapeDtypeStruct((n, H), F32),
            jax.ShapeDtypeStruct((n, H), F32),
        ],
    )(f2d, mp['w1'], mp['b1'].reshape(1, H), mp['w2'], mp['b2'].reshape(1, H))


def _node_body(x_ref, wcat_ref, att_ref, xlr_ref, s_ref):
    xlr = jnp.dot(x_ref[...], wcat_ref[...], preferred_element_type=F32)
    xlr_ref[...] = xlr
    m = _lr(xlr[:, :HH] + xlr[:, HH:]) * att_ref[...]
    s_ref[...] = jnp.dot(m, _hsel(), preferred_element_type=F32)


def _node(x, wcat, att):
    n = x.shape[0]
    return _pc(
        _node_body,
        grid=(n // NTILE,),
        in_specs=[
            pl.BlockSpec((NTILE, H), lambda i: (i, 0)),
            pl.BlockSpec((H, 2 * HH), lambda i: (0, 0)),
            pl.BlockSpec((1, HH), lambda i: (0, 0)),
        ],
        out_specs=[
            pl.BlockSpec((NTILE, 2 * HH), lambda i: (i, 0)),
            pl.BlockSpec((NTILE, HEADS), lambda i: (i, 0)),
        ],
        out_shape=[
            jax.ShapeDtypeStruct((n, 2 * HH), F32),
            jax.ShapeDtypeStruct((n, HEADS), F32),
        ],
    )(x, wcat, att)


def _edge_body(idxc_ref, txlr_ref, rc_ref, sc_ref, xc_ref,
               att_ref, bias_ref, g_ref, bb_ref, xout_ref, lct_ref):
    idxc = idxc_ref[...]
    lane = jax.lax.broadcasted_iota(jnp.int32, (CT_E, NT), 1).astype(F32)
    P = (idxc == lane).astype(F32)
    G = jnp.dot(P, txlr_ref[...], preferred_element_type=F32)
    rc = rc_ref[...]
    xl_t = G[:, :HH]
    xr_t = G[:, HH:]
    xl_c = rc[:, :HH]
    xr_c = rc[:, HH:]
    att = att_ref[...]
    hs = _hsel()
    l_tc = jnp.dot(_lr(xl_t + xr_c) * att, hs, preferred_element_type=F32)
    l_ct = jnp.dot(_lr(xl_c + xr_t) * att, hs, preferred_element_type=F32)
    lct_ref[...] = l_ct
    sc4 = sc_ref[...]
    lmax = jnp.maximum(l_tc, sc4)
    e1 = jnp.exp(l_tc - lmax)
    e2 = jnp.exp(sc4 - lmax)
    den = e1 + e2 + 1e-16
    a1 = e1 / den
    a2 = e2 / den
    conv = jnp.zeros((CT_E, H), F32)
    for h in range(HEADS):
        sl = slice(h * H, (h + 1) * H)
        conv += a1[:, h:h + 1] * xl_t[:, sl] + a2[:, h:h + 1] * xl_c[:, sl]
    conv = conv * 0.25 + bias_ref[...]
    xout_ref[...] = _layernorm_relu_add(conv, xc_ref[...], g_ref[...], bb_ref[...])


def _edge(idxc, txlr, rc, sc4, xc, att, bias, g, bb):
    rep = lambda i: (0, 0)
    return _pc(
        _edge_body,
        grid=(NC // CT_E,),
        in_specs=[
            pl.BlockSpec((CT_E, 1), lambda i: (i, 0)),
            pl.BlockSpec((NT, 2 * HH), rep),
            pl.BlockSpec((CT_E, 2 * HH), lambda i: (i, 0)),
            pl.BlockSpec((CT_E, HEADS), lambda i: (i, 0)),
            pl.BlockSpec((CT_E, H), lambda i: (i, 0)),
            pl.BlockSpec((1, HH), rep),
            pl.BlockSpec((1, H), rep),
            pl.BlockSpec((1, H), rep),
            pl.BlockSpec((1, H), rep),
        ],
        out_specs=[
            pl.BlockSpec((CT_E, H), lambda i: (i, 0)),
            pl.BlockSpec((CT_E, HEADS), lambda i: (i, 0)),
        ],
        out_shape=[
            jax.ShapeDtypeStruct((NC, H), F32),
            jax.ShapeDtypeStruct((NC, HEADS), F32),
        ],
    )(idxc, txlr, rc, sc4, xc, att, bias, g, bb)


def _segmax_body(idxr_ref, lctt_ref, ts_ref, maxt_ref, macc):
    k = pl.program_id(0)

    @pl.when(k == 0)
    def _():
        macc[...] = ts_ref[...]

    idxr = idxr_ref[...]
    rows = jax.lax.broadcasted_iota(jnp.int32, (NT, CT_S), 0).astype(F32)
    hit = rows == idxr
    lctt = lctt_ref[...]
    for h in range(HEADS):
        row = lctt[h:h + 1, :]
        masked = jnp.where(hit, row, -jnp.inf)
        macc[:, h:h + 1] = jnp.maximum(
            macc[:, h:h + 1], jnp.max(masked, axis=1, keepdims=True))

    @pl.when(k == pl.num_programs(0) - 1)
    def _():
        maxt_ref[...] = macc[...]


def _segmax(idxr, lctt, ts4):
    rep = lambda i: (0, 0)
    return _pc(
        _segmax_body,
        grid=(NC // CT_S,),
        in_specs=[
            pl.BlockSpec((1, CT_S), lambda i: (0, i)),
            pl.BlockSpec((HEADS, CT_S), lambda i: (0, i)),
            pl.BlockSpec((NT, HEADS), rep),
        ],
        out_specs=pl.BlockSpec((NT, HEADS), rep),
        out_shape=jax.ShapeDtypeStruct((NT, HEADS), F32),
        scratch_shapes=[pltpu.VMEM((NT, HEADS), F32)],
    )(idxr, lctt, ts4)


def _scat_body(idxr_ref, idxc_ref, lct_ref, xlc_ref, maxt_ref, ts_ref,
               txl_ref, xt_ref, bias_ref, g_ref, bb_ref, xout_ref, acc, accex):
    k = pl.program_id(0)

    @pl.when(k == 0)
    def _():
        acc[...] = jnp.zeros_like(acc)
        accex[...] = jnp.zeros_like(accex)

    idxc = idxc_ref[...]
    lane = jax.lax.broadcasted_iota(jnp.int32, (CT_S, NT), 1).astype(F32)
    P = (idxc == lane).astype(F32)
    mt_c = jnp.dot(P, maxt_ref[...], preferred_element_type=F32)
    ex = jnp.exp(lct_ref[...] - mt_c)
    xlc = xlc_ref[...]
    w = jnp.concatenate(
        [ex[:, h:h + 1] * xlc[:, h * H:(h + 1) * H] for h in range(HEADS)], axis=1)
    idxr = idxr_ref[...]
    rows = jax.lax.broadcasted_iota(jnp.int32, (NT, CT_S), 0).astype(F32)
    P_T = (rows == idxr).astype(F32)
    acc[...] += jnp.dot(P_T, w, preferred_element_type=F32)
    accex[...] += jnp.dot(P_T, ex, preferred_element_type=F32)

    @pl.when(k == pl.num_programs(0) - 1)
    def _():
        txl = txl_ref[...]
        a = acc[...]
        selfex = jnp.exp(ts_ref[...] - maxt_ref[...])
        ae = accex[...] + selfex + 1e-16
        conv = jnp.zeros((NT, H), F32)
        for h in range(HEADS):
            sl = slice(h * H, (h + 1) * H)
            conv += (a[:, sl] + selfex[:, h:h + 1] * txl[:, sl]) / ae[:, h:h + 1]
        conv = conv * 0.25 + bias_ref[...]
        xout_ref[...] = _layernorm_relu_add(conv, xt_ref[...], g_ref[...], bb_ref[...])


def _scat(idxr, idxc, lct4, xlc, maxt, ts4, txl, xt, bias, g, bb):
    rep = lambda i: (0, 0)
    return _pc(
        _scat_body,
        grid=(NC // CT_S,),
        in_specs=[
            pl.BlockSpec((1, CT_S), lambda i: (0, i)),
            pl.BlockSpec((CT_S, 1), lambda i: (i, 0)),
            pl.BlockSpec((CT_S, HEADS), lambda i: (i, 0)),
            pl.BlockSpec((CT_S, HH), lambda i: (i, 0)),
            pl.BlockSpec((NT, HEADS), rep),
            pl.BlockSpec((NT, HEADS), rep),
            pl.BlockSpec((NT, HH), rep),
            pl.BlockSpec((NT, H), rep),
            pl.BlockSpec((1, H), rep),
            pl.BlockSpec((1, H), rep),
            pl.BlockSpec((1, H), rep),
        ],
        out_specs=pl.BlockSpec((NT, H), rep),
        out_shape=jax.ShapeDtypeStruct((NT, H), F32),
        scratch_shapes=[
            pltpu.VMEM((NT, HH), F32),
            pltpu.VMEM((NT, HEADS), F32),
        ],
    )(idxr, idxc, lct4, xlc, maxt, ts4, txl, xt, bias, g, bb)


def kernel(cell_feats, topo_feats, cell_mask, topo_mask, params):
    cf = jnp.transpose(cell_feats, (0, 2, 1)).reshape(BS * NC, FC)
    tf = jnp.transpose(topo_feats, (0, 2, 1)).reshape(BS * NT, FT)
    cell_emb, pos_c = _embed(cf, params['cell_mlp'], 1024)
    topo_emb, pos_t = _embed(tf, params['topo_mlp'], 512)
    idx_f = cf[:, FC - 1].reshape(BS, NC)
    xc = cell_emb.reshape(BS, NC, H)
    xt = topo_emb.reshape(BS, NT, H)
    for p in params['gnn']:
        wcat = jnp.concatenate([p['w_src'], p['w_dst']], axis=1)
        att = p['att'].reshape(1, HH)
        bias = p['bias'].reshape(1, H)
        g = p['ln_g'].reshape(1, H)
        bb = p['ln_b'].reshape(1, H)
        nxc, nxt = [], []
        for b in range(BS):
            x_b = jnp.concatenate([xc[b], xt[b]], axis=0)
            xlr, s4 = _node(x_b, wcat, att)
            txlr = xlr[NC:]
            ts4 = s4[NC:]
            idxc = idx_f[b].reshape(NC, 1)
            idxr = idx_f[b].reshape(1, NC)
            xout_c, lct4 = _edge(idxc, txlr, xlr[:NC], s4[:NC],
                                 xc[b], att, bias, g, bb)
            maxt = _segmax(idxr, lct4.T, ts4)
            xout_t = _scat(idxr, idxc, lct4, xlr[:NC, :HH], maxt, ts4,
                           txlr[:, :HH], xt[b], bias, g, bb)
            nxc.append(xout_c)
            nxt.append(xout_t)
        xc = jnp.stack(nxc)
        xt = jnp.stack(nxt)
    return (xc, xt, pos_c.reshape(BS, NC, H), pos_t.reshape(BS, NT, H))


# speedup vs baseline: 25.9154x; 1.1176x over previous
import functools

import jax
import jax.numpy as jnp
from jax import lax
from jax.experimental import pallas as pl
from jax.experimental.pallas import tpu as pltpu
from jax.experimental.pallas import tpu_sc as plsc

BS = 2
NC = 8192
NT = 512
FC = 16
FT = 10
H = 128
HEADS = 4
NPF = 32
NN = NC + NT
NTOT = BS * NN
HH = HEADS * H
F32 = jnp.float32

CT = 512
CBLK = NC // CT
NBLK = NN // CT
B_ALL = BS * NC

NWORK = 32
BPW = B_ALL // NWORK
GCH = 64


def _pc(body, **kw):
    return pl.pallas_call(body, **kw)


def _lr(x):
    return jnp.where(x > 0, x, 0.2 * x)


def _hsel():
    d = jax.lax.broadcasted_iota(jnp.int32, (HH, HEADS), 0) // H
    hcol = jax.lax.broadcasted_iota(jnp.int32, (HH, HEADS), 1)
    return (d == hcol).astype(F32)


def _layernorm_relu_add(conv, x, g, b):
    mu = jnp.mean(conv, axis=1, keepdims=True)
    cmu = conv - mu
    var = jnp.mean(cmu * cmu, axis=1, keepdims=True)
    y = cmu * jax.lax.rsqrt(var + 1e-5) * g + b
    return x + jnp.maximum(y, 0.0)


_SC_GATHER_FN = None


def _build_sc_gather():
    mesh = plsc.VectorSubcoreMesh(core_axis_name="c", subcore_axis_name="s")

    @functools.partial(
        pl.kernel, mesh=mesh,
        out_type=jax.ShapeDtypeStruct((B_ALL, 2 * HH), jnp.float32),
        scratch_types=[
            pltpu.VMEM((GCH,), jnp.int32),
            pltpu.VMEM((GCH, 2 * HH), jnp.float32),
            pltpu.SemaphoreType.DMA,
        ],
    )
    def gather_kernel(table_hbm, gidx_hbm, out_hbm, idx_v, rows_v, sem):
        cid = lax.axis_index("c")
        sid = lax.axis_index("s")
        wid = sid * 2 + cid
        base = wid * BPW
        for ch in range(BPW // GCH):
            off = base + ch * GCH
            pltpu.sync_copy(gidx_hbm.at[pl.ds(off, GCH)], idx_v)
            pltpu.async_copy(table_hbm.at[idx_v], rows_v, sem).wait()
            pltpu.sync_copy(rows_v, out_hbm.at[pl.ds(off, GCH)])

    return gather_kernel


def _sc_gather(table, gidx):
    global _SC_GATHER_FN
    if _SC_GATHER_FN is None:
        _SC_GATHER_FN = _build_sc_gather()
    return _SC_GATHER_FN(table, gidx)


def _embed_body(f_ref, w1_ref, b1_ref, w2_ref, b2_ref, emb_ref, pos_ref):
    f = f_ref[...]
    h = jnp.maximum(jnp.dot(f, w1_ref[...], preferred_element_type=F32) + b1_ref[...], 0.0)
    emb_ref[...] = jnp.dot(h, w2_ref[...], preferred_element_type=F32) + b2_ref[...]
    i = jax.lax.broadcasted_iota(jnp.int32, (1, NPF), 1).astype(F32)
    dim_t = jnp.exp((2.0 * jnp.floor(i / 2.0) / NPF) * jnp.log(10000.0))
    pe = f[:, 4:5] / dim_t
    pp = f[:, 5:6] / dim_t
    pos_ref[:, 0:NPF] = jnp.sin(pe)
    pos_ref[:, NPF:2 * NPF] = jnp.cos(pe)
    pos_ref[:, 2 * NPF:3 * NPF] = jnp.sin(pp)
    pos_ref[:, 3 * NPF:4 * NPF] = jnp.cos(pp)


def _embed(f2d, mp, tile):
    n, fdim = f2d.shape
    return _pc(
        _embed_body,
        grid=(n // tile,),
        in_specs=[
            pl.BlockSpec((tile, fdim), lambda i: (i, 0)),
            pl.BlockSpec((fdim, H), lambda i: (0, 0)),
            pl.BlockSpec((1, H), lambda i: (0, 0)),
            pl.BlockSpec((H, H), lambda i: (0, 0)),
            pl.BlockSpec((1, H), lambda i: (0, 0)),
        ],
        out_specs=[
            pl.BlockSpec((tile, H), lambda i: (i, 0)),
            pl.BlockSpec((tile, H), lambda i: (i, 0)),
        ],
        out_shape=[
            jax.ShapeDtypeStruct((n, H), F32),
            jax.ShapeDtypeStruct((n, H), F32),
        ],
    )(f2d, mp['w1'], mp['b1'].reshape(1, H), mp['w2'], mp['b2'].reshape(1, H))


def _node_body(x_ref, wcat_ref, att_ref, xlr_ref, s_ref):
    xlr = jnp.dot(x_ref[...], wcat_ref[...], preferred_element_type=F32)
    xlr_ref[...] = xlr
    m = _lr(xlr[:, :HH] + xlr[:, HH:]) * att_ref[...]
    s_ref[...] = jnp.dot(m, _hsel(), preferred_element_type=F32)


def _node(x, wcat, att):
    return _pc(
        _node_body,
        grid=(NTOT // CT,),
        in_specs=[
            pl.BlockSpec((CT, H), lambda i: (i, 0)),
            pl.BlockSpec((H, 2 * HH), lambda i: (0, 0)),
            pl.BlockSpec((1, HH), lambda i: (0, 0)),
        ],
        out_specs=[
            pl.BlockSpec((CT, 2 * HH), lambda i: (i, 0)),
            pl.BlockSpec((CT, HEADS), lambda i: (i, 0)),
        ],
        out_shape=[
            jax.ShapeDtypeStruct((NTOT, 2 * HH), F32),
            jax.ShapeDtypeStruct((NTOT, HEADS), F32),
        ],
    )(x, wcat, att)


def _edge_body(g_ref, rc_ref, sc_ref, xc_ref,
               att_ref, bias_ref, gg_ref, bb_ref, xout_ref, lct_ref):
    gv = g_ref[...]
    rc = rc_ref[...]
    xl_t = gv[:, :HH]
    xr_t = gv[:, HH:]
    xl_c = rc[:, :HH]
    xr_c = rc[:, HH:]
    att = att_ref[...]
    hs = _hsel()
    l_tc = jnp.dot(_lr(xl_t + xr_c) * att, hs, preferred_element_type=F32)
    l_ct = jnp.dot(_lr(xl_c + xr_t) * att, hs, preferred_element_type=F32)
    lct_ref[...] = l_ct
    sc4 = sc_ref[...]
    lmax = jnp.maximum(l_tc, sc4)
    e1 = jnp.exp(l_tc - lmax)
    e2 = jnp.exp(sc4 - lmax)
    den = e1 + e2 + 1e-16
    a1 = e1 / den
    a2 = e2 / den
    conv = jnp.zeros((CT, H), F32)
    for h in range(HEADS):
        sl = slice(h * H, (h + 1) * H)
        conv += a1[:, h:h + 1] * xl_t[:, sl] + a2[:, h:h + 1] * xl_c[:, sl]
    conv = conv * 0.25 + bias_ref[...]
    xout_ref[...] = _layernorm_relu_add(conv, xc_ref[...], gg_ref[...], bb_ref[...])


def _edge(gth, xlr, s4, x, att, bias, g, bb):
    rep = lambda b, j: (0, 0)
    cmap = lambda b, j: (b * CBLK + j, 0)
    nmap = lambda b, j: (b * NBLK + j, 0)
    return _pc(
        _edge_body,
        grid=(BS, CBLK),
        in_specs=[
            pl.BlockSpec((CT, 2 * HH), cmap),
            pl.BlockSpec((CT, 2 * HH), nmap),
            pl.BlockSpec((CT, HEADS), nmap),
            pl.BlockSpec((CT, H), nmap),
            pl.BlockSpec((1, HH), rep),
            pl.BlockSpec((1, H), rep),
            pl.BlockSpec((1, H), rep),
            pl.BlockSpec((1, H), rep),
        ],
        out_specs=[
            pl.BlockSpec((CT, H), cmap),
            pl.BlockSpec((CT, HEADS), cmap),
        ],
        out_shape=[
            jax.ShapeDtypeStruct((B_ALL, H), F32),
            jax.ShapeDtypeStruct((B_ALL, HEADS), F32),
        ],
    )(gth, xlr, s4, x, att, bias, g, bb)


def _segmax_body(idxr_ref, lctt_ref, ts_ref, maxt_ref, macc):
    j = pl.program_id(1)

    @pl.when(j == 0)
    def _():
        macc[...] = ts_ref[...]

    idxr = idxr_ref[0]
    rows = jax.lax.broadcasted_iota(jnp.int32, (NT, CT), 0).astype(F32)
    hit = rows == idxr
    lctt = lctt_ref[...]
    for h in range(HEADS):
        row = lctt[h:h + 1, :]
        masked = jnp.where(hit, row, -jnp.inf)
        macc[:, h:h + 1] = jnp.maximum(
            macc[:, h:h + 1], jnp.max(masked, axis=1, keepdims=True))

    @pl.when(j == pl.num_programs(1) - 1)
    def _():
        maxt_ref[...] = macc[...]


def _segmax(idx_f, lctt, s4):
    return _pc(
        _segmax_body,
        grid=(BS, CBLK),
        in_specs=[
            pl.BlockSpec((1, 1, CT), lambda b, j: (b, 0, j)),
            pl.BlockSpec((HEADS, CT), lambda b, j: (0, b * CBLK + j)),
            pl.BlockSpec((NT, HEADS), lambda b, j: (b * NBLK + CBLK, 0)),
        ],
        out_specs=pl.BlockSpec((NT, HEADS), lambda b, j: (b, 0)),
        out_shape=jax.ShapeDtypeStruct((BS * NT, HEADS), F32),
        scratch_shapes=[pltpu.VMEM((NT, HEADS), F32)],
    )(idx_f, lctt, s4)


def _scat_body(idxr_ref, idxc_ref, lct_ref, xlr_ref, maxt_ref, ts_ref,
               xt_ref, bias_ref, gg_ref, bb_ref, xout_ref, acc, accex):
    j = pl.program_id(1)

    @pl.when(j == 0)
    def _():
        acc[...] = jnp.zeros_like(acc)
        accex[...] = jnp.zeros_like(accex)

    @pl.when(j < CBLK)
    def _():
        idxc = idxc_ref[...]
        lane = jax.lax.broadcasted_iota(jnp.int32, (CT, NT), 1).astype(F32)
        P = (idxc == lane).astype(F32)
        mt_c = jnp.dot(P, maxt_ref[...], preferred_element_type=F32)
        ex = jnp.exp(lct_ref[...] - mt_c)
        xlc = xlr_ref[...][:, :HH]
        w = jnp.concatenate(
            [ex[:, h:h + 1] * xlc[:, h * H:(h + 1) * H] for h in range(HEADS)], axis=1)
        idxr = idxr_ref[0]
        rows = jax.lax.broadcasted_iota(jnp.int32, (NT, CT), 0).astype(F32)
        P_T = (rows == idxr).astype(F32)
        acc[...] += jnp.dot(P_T, w, preferred_element_type=F32)
        accex[...] += jnp.dot(P_T, ex, preferred_element_type=F32)

    @pl.when(j == CBLK)
    def _():
        txl = xlr_ref[...][:, :HH]
        a = acc[...]
        selfex = jnp.exp(ts_ref[...] - maxt_ref[...])
        ae = accex[...] + selfex + 1e-16
        conv = jnp.zeros((NT, H), F32)
        for h in range(HEADS):
            sl = slice(h * H, (h + 1) * H)
            conv += (a[:, sl] + selfex[:, h:h + 1] * txl[:, sl]) / ae[:, h:h + 1]
        conv = conv * 0.25 + bias_ref[...]
        xout_ref[...] = _layernorm_relu_add(conv, xt_ref[...], gg_ref[...], bb_ref[...])


def _scat(idx_f, idxc2d, lct4, xlr, maxt, s4, x, bias, g, bb):
    rep = lambda b, j: (0, 0)
    cmap = lambda b, j: (b * CBLK + jnp.minimum(j, CBLK - 1), 0)
    rmap = lambda b, j: (b, 0, jnp.minimum(j, CBLK - 1))
    tmap = lambda b, j: (b * NBLK + CBLK, 0)
    xlrmap = lambda b, j: (b * NBLK + j, 0)
    return _pc(
        _scat_body,
        grid=(BS, CBLK + 1),
        in_specs=[
            pl.BlockSpec((1, 1, CT), rmap),
            pl.BlockSpec((CT, 1), cmap),
            pl.BlockSpec((CT, HEADS), cmap),
            pl.BlockSpec((CT, 2 * HH), xlrmap),
            pl.BlockSpec((NT, HEADS), lambda b, j: (b, 0)),
            pl.BlockSpec((NT, HEADS), tmap),
            pl.BlockSpec((NT, H), tmap),
            pl.BlockSpec((1, H), rep),
            pl.BlockSpec((1, H), rep),
            pl.BlockSpec((1, H), rep),
        ],
        out_specs=pl.BlockSpec((NT, H), lambda b, j: (b, 0)),
        out_shape=jax.ShapeDtypeStruct((BS * NT, H), F32),
        scratch_shapes=[
            pltpu.VMEM((NT, HH), F32),
            pltpu.VMEM((NT, HEADS), F32),
        ],
    )(idx_f, idxc2d, lct4, xlr, maxt, s4, x, bias, g, bb)


def kernel(cell_feats, topo_feats, cell_mask, topo_mask, params):
    cf = jnp.transpose(cell_feats, (0, 2, 1)).reshape(BS * NC, FC)
    tf = jnp.transpose(topo_feats, (0, 2, 1)).reshape(BS * NT, FT)
    cell_emb, pos_c = _embed(cf, params['cell_mlp'], 1024)
    topo_emb, pos_t = _embed(tf, params['topo_mlp'], 512)
    idx_f = cf[:, FC - 1].reshape(BS, 1, NC)
    idxc2d = idx_f.reshape(B_ALL, 1)
    idx_i = idx_f.reshape(BS, NC).astype(jnp.int32)
    offs = (jnp.arange(BS, dtype=jnp.int32) * NN)[:, None]
    gidx = (idx_i + NC + offs).reshape(-1)
    x = jnp.concatenate([
        cell_emb[:NC], topo_emb[:NT], cell_emb[NC:], topo_emb[NT:]], axis=0)
    for p in params['gnn']:
        wcat = jnp.concatenate([p['w_src'], p['w_dst']], axis=1)
        att = p['att'].reshape(1, HH)
        bias = p['bias'].reshape(1, H)
        g = p['ln_g'].reshape(1, H)
        bb = p['ln_b'].reshape(1, H)
        xlr, s4 = _node(x, wcat, att)
        gth = _sc_gather(xlr, gidx)
        xout_c, lct4 = _edge(gth, xlr, s4, x, att, bias, g, bb)
        maxt = _segmax(idx_f, lct4.T, s4)
        xout_t = _scat(idx_f, idxc2d, lct4, xlr, maxt, s4, x, bias, g, bb)
        x = jnp.concatenate([
            xout_c[:NC], xout_t[:NT], xout_c[NC:], xout_t[NT:]], axis=0)
        last_c, last_t = xout_c, xout_t
    return (last_c.reshape(BS, NC, H), last_t.reshape(BS, NT, H),
            pos_c.reshape(BS, NC, H), pos_t.reshape(BS, NT, H))
